# baseline (speedup 1.0000x reference)
import functools
import math

import jax
import jax.numpy as jnp
from jax import lax
from jax.experimental import pallas as pl
from jax.experimental.pallas import tpu as pltpu
from jax.experimental.pallas import tpu_sc as plsc

_info = plsc.get_sparse_core_info()
_NC, _NS, _L = _info.num_cores, _info.num_subcores, _info.num_lanes
_NW = _NC * _NS


@functools.lru_cache(maxsize=None)
def _build(batch: int, seq: int, d: int):
    s_per_w = seq // _NW
    cp = 16
    npb = s_per_w // cp
    nch = npb * batch
    nb = 5
    la = 3
    assert seq % _NW == 0 and s_per_w % cp == 0 and d % _L == 0
    scale = math.sqrt(d)
    mesh = plsc.VectorSubcoreMesh(core_axis_name="c", subcore_axis_name="s")

    @functools.partial(
        pl.kernel,
        mesh=mesh,
        out_type=jax.ShapeDtypeStruct((batch * seq, d), jnp.float32),
        scratch_types=[
            pltpu.VMEM((batch * s_per_w,), jnp.int32),
            pltpu.VMEM((nb, cp, d), jnp.float32),
            pltpu.VMEM((2, cp, d), jnp.float32),
            pltpu.SemaphoreType.DMA((nb,)),
            pltpu.SemaphoreType.DMA((2,)),
            pltpu.SemaphoreType.DMA((nb,)),
            pltpu.SemaphoreType.DMA,
        ],
    )
    def emb(x_hbm, tok_hbm, pos_hbm, out_hbm,
            idx_v, tok_v, pos_v, sem_g, sem_p, sem_o, sem_i):
        wid = lax.axis_index("s") * _NC + lax.axis_index("c")
        pos_lo = wid * s_per_w

        def gather_start(c):
            pb, bb = divmod(c, batch)
            slot = c % nb
            return pltpu.async_copy(
                tok_hbm.at[idx_v.at[pl.ds(bb * s_per_w + pb * cp, cp)]],
                tok_v.at[slot], sem_g.at[slot])

        def pos_start(pb):
            p = pb % 2
            return pltpu.async_copy(
                pos_hbm.at[pl.ds(pos_lo + pb * cp, cp)], pos_v.at[p],
                sem_p.at[p])

        def out_start(c):
            pb, bb = divmod(c, batch)
            slot = c % nb
            return pltpu.async_copy(
                tok_v.at[slot],
                out_hbm.at[pl.ds(bb * seq + pos_lo + pb * cp, cp)],
                sem_o.at[slot])

        def compute(slot, p):
            @plsc.parallel_loop(0, d, step=_L, unroll=2)
            def _(o):
                sl = pl.ds(o, _L)
                for r in range(cp):
                    tok_v[slot, r, sl] = (
                        tok_v[slot, r, sl] * scale + pos_v[p, r, sl])

        h_g = [None] * nb
        h_p = [None] * 2
        h_o = [None] * nb
        h_p[0] = pos_start(0)
        if npb > 1:
            h_p[1] = pos_start(1)
        h_i = [
            pltpu.async_copy(
                x_hbm.at[pl.ds(bb * seq + pos_lo, s_per_w)],
                idx_v.at[pl.ds(bb * s_per_w, s_per_w)], sem_i)
            for bb in range(batch)
        ]
        for h in h_i:
            h.wait()
        for c in range(min(la, nch)):
            h_g[c % nb] = gather_start(c)
        for c in range(nch):
            slot = c % nb
            pb, bb = divmod(c, batch)
            if c + la < nch:
                gs = (c + la) % nb
                if h_o[gs] is not None:
                    h_o[gs].wait()
                    h_o[gs] = None
                h_g[gs] = gather_start(c + la)
            h_g[slot].wait()
            if bb == 0 and h_p[pb % 2] is not None:
                h_p[pb % 2].wait()
                h_p[pb % 2] = None
            compute(slot, pb % 2)
            if bb == batch - 1 and pb + 2 < npb:
                h_p[pb % 2] = pos_start(pb + 2)
            h_o[slot] = out_start(c)
        for slot in range(nb):
            if h_o[slot] is not None:
                h_o[slot].wait()

    return emb


def kernel(x, token_table, pos_table):
    batch, seq = x.shape
    d = token_table.shape[1]
    emb = _build(batch, seq, d)
    out = emb(x.reshape(-1).astype(jnp.int32), token_table, pos_table)
    return out.reshape(batch, seq, d)


# speedup vs baseline: 1.1551x; 1.1551x over previous
import functools
import math

import jax
import jax.numpy as jnp
from jax import lax
from jax.experimental import pallas as pl
from jax.experimental.pallas import tpu as pltpu
from jax.experimental.pallas import tpu_sc as plsc

_info = plsc.get_sparse_core_info()
_NC, _NS, _L = _info.num_cores, _info.num_subcores, _info.num_lanes
_NW = _NC * _NS


@functools.lru_cache(maxsize=None)
def _build(batch: int, seq: int, d: int):
    s_per_w = seq // _NW
    cp = 8
    npb = s_per_w // cp
    assert seq % _NW == 0 and s_per_w % cp == 0 and d % _L == 0
    scale = math.sqrt(d)
    mesh = plsc.VectorSubcoreMesh(core_axis_name="c", subcore_axis_name="s")

    @functools.partial(
        pl.kernel,
        mesh=mesh,
        out_type=jax.ShapeDtypeStruct((batch * seq, d), jnp.float32),
        scratch_types=[
            pltpu.VMEM((batch * s_per_w,), jnp.int32),
            pltpu.VMEM((3, batch, cp, d), jnp.float32),
            pltpu.VMEM((2, cp, d), jnp.float32),
            pltpu.SemaphoreType.DMA((3, batch)),
            pltpu.SemaphoreType.DMA((2,)),
            pltpu.SemaphoreType.DMA((3, batch)),
            pltpu.SemaphoreType.DMA,
        ],
    )
    def emb(x_hbm, tok_hbm, pos_hbm, out_hbm,
            idx_v, tok_v, pos_v, sem_g, sem_p, sem_o, sem_i):
        wid = lax.axis_index("s") * _NC + lax.axis_index("c")
        pos_lo = wid * s_per_w

        def gather_start(pb):
            g = pb % 3
            return [
                pltpu.async_copy(
                    tok_hbm.at[idx_v.at[pl.ds(bb * s_per_w + pb * cp, cp)]],
                    tok_v.at[g, bb], sem_g.at[g, bb])
                for bb in range(batch)
            ]

        def pos_start(pb):
            p = pb % 2
            return pltpu.async_copy(
                pos_hbm.at[pl.ds(pos_lo + pb * cp, cp)], pos_v.at[p],
                sem_p.at[p])

        def out_start(pb):
            g = pb % 3
            return [
                pltpu.async_copy(
                    tok_v.at[g, bb],
                    out_hbm.at[pl.ds(bb * seq + pos_lo + pb * cp, cp)],
                    sem_o.at[g, bb])
                for bb in range(batch)
            ]

        def compute(g, p):
            @plsc.parallel_loop(0, d, step=_L)
            def _(o):
                sl = pl.ds(o, _L)
                for i in range(cp):
                    pv = pos_v[p, i, sl]
                    for bb in range(batch):
                        tok_v[g, bb, i, sl] = tok_v[g, bb, i, sl] * scale + pv

        h_p = [None] * 2
        h_p[0] = pos_start(0)
        if npb > 1:
            h_p[1] = pos_start(1)
        h_i = [
            pltpu.async_copy(
                x_hbm.at[pl.ds(bb * seq + pos_lo, s_per_w)],
                idx_v.at[pl.ds(bb * s_per_w, s_per_w)], sem_i)
            for bb in range(batch)
        ]
        for h in h_i:
            h.wait()

        h_g = [None] * 3
        h_o = [None] * 3
        h_g[0] = gather_start(0)
        for pb in range(npb):
            g = pb % 3
            if pb + 1 < npb:
                gn = (pb + 1) % 3
                if h_o[gn] is not None:
                    for h in h_o[gn]:
                        h.wait()
                    h_o[gn] = None
                h_g[gn] = gather_start(pb + 1)
            for h in h_g[g]:
                h.wait()
            p = pb % 2
            if h_p[p] is not None:
                h_p[p].wait()
                h_p[p] = None
            compute(g, p)
            if pb + 2 < npb:
                h_p[p] = pos_start(pb + 2)
            h_o[g] = out_start(pb)
        for g in range(3):
            if h_o[g] is not None:
                for h in h_o[g]:
                    h.wait()

    return emb


def kernel(x, token_table, pos_table):
    batch, seq = x.shape
    d = token_table.shape[1]
    emb = _build(batch, seq, d)
    out = emb(x.reshape(-1).astype(jnp.int32), token_table, pos_table)
    return out.reshape(batch, seq, d)
